# Initial kernel scaffold; baseline (speedup 1.0000x reference)
import functools

import jax
import jax.numpy as jnp
from jax import lax
from jax.experimental import pallas as pl
from jax.experimental.pallas import tpu as pltpu
from jax.experimental.pallas import tpu_sc as plsc

B, L, D = 4096, 200, 64
N = B * L
NC, NS = 2, 16
NW = NC * NS
PER_W = N // NW
C = 128
CHUNKS = PER_W // C

_mesh = plsc.VectorSubcoreMesh(core_axis_name="c", subcore_axis_name="s")


@functools.partial(
    pl.kernel,
    mesh=_mesh,
    out_type=jax.ShapeDtypeStruct((N, D), jnp.float32),
    scratch_types=[
        pltpu.VMEM((C,), jnp.int32),
        pltpu.VMEM((C,), jnp.int32),
        pltpu.VMEM((C, D), jnp.float32),
        pltpu.VMEM((C, D), jnp.float32),
        pltpu.SemaphoreType.DMA,
        pltpu.SemaphoreType.DMA,
    ],
)
def _sol_embedding(t_hbm, v_hbm, tt_hbm, vt_hbm, out_hbm,
                   ti, vi, bufv, buft, sem_v, sem_t):
    wid = lax.axis_index("s") * NC + lax.axis_index("c")
    base = wid * PER_W

    def chunk_body(g, carry):
        off = base + g * C
        pltpu.sync_copy(t_hbm.at[pl.ds(off, C)], ti)
        pltpu.sync_copy(v_hbm.at[pl.ds(off, C)], vi)
        cp_v = pltpu.async_copy(vt_hbm.at[vi], bufv, sem_v)
        cp_t = pltpu.async_copy(tt_hbm.at[ti], buft, sem_t)
        cp_v.wait()
        cp_t.wait()

        def add_body(i, c2):
            for j in range(D // 16):
                x = buft[i, pl.ds(j * 16, 16)]
                plsc.addupdate(bufv.at[i, pl.ds(j * 16, 16)], x)
            return c2

        lax.fori_loop(0, C, add_body, 0)
        pltpu.sync_copy(bufv, out_hbm.at[pl.ds(off, C)])
        return carry

    lax.fori_loop(0, CHUNKS, chunk_body, 0)


def kernel(t, v, type_table, value_table):
    t_flat = jnp.reshape(t.astype(jnp.int32), (N,))
    v_flat = jnp.reshape(v.astype(jnp.int32), (N,))
    out = _sol_embedding(t_flat, v_flat, type_table, value_table)
    return jnp.reshape(out, (B, L, D))


# speedup vs baseline: 2.0538x; 2.0538x over previous
import functools

import jax
import jax.numpy as jnp
from jax import lax
from jax.experimental import pallas as pl
from jax.experimental.pallas import tpu as pltpu
from jax.experimental.pallas import tpu_sc as plsc

B, L, D = 4096, 200, 64
N = B * L
NC, NS = 2, 16
NW = NC * NS
PER_W = N // NW
C = 128
CHUNKS = PER_W // C

_mesh = plsc.VectorSubcoreMesh(core_axis_name="c", subcore_axis_name="s")


@functools.partial(
    pl.kernel,
    mesh=_mesh,
    out_type=jax.ShapeDtypeStruct((N, D), jnp.float32),
    compiler_params=pltpu.CompilerParams(use_tc_tiling_on_sc=False),
    scratch_types=[
        pltpu.VMEM((C,), jnp.int32),
        pltpu.VMEM((C,), jnp.int32),
        pltpu.VMEM((C, D), jnp.float32),
        pltpu.VMEM((C, D), jnp.float32),
        pltpu.SemaphoreType.DMA,
        pltpu.SemaphoreType.DMA,
    ],
)
def _sol_embedding(t_hbm, v_hbm, tt_hbm, vt_hbm, out_hbm,
                   ti, vi, bufv, buft, sem_v, sem_t):
    wid = lax.axis_index("s") * NC + lax.axis_index("c")
    base = wid * PER_W

    def chunk_body(g, carry):
        off = base + g * C
        pltpu.sync_copy(t_hbm.at[pl.ds(off, C)], ti)
        pltpu.sync_copy(v_hbm.at[pl.ds(off, C)], vi)
        cp_v = pltpu.async_copy(vt_hbm.at[vi], bufv, sem_v)
        cp_t = pltpu.async_copy(tt_hbm.at[ti], buft, sem_t)
        cp_v.wait()
        cp_t.wait()

        def add_body(i, c2):
            for j in range(D // 16):
                x = buft[i, pl.ds(j * 16, 16)]
                plsc.addupdate(bufv.at[i, pl.ds(j * 16, 16)], x)
            return c2

        lax.fori_loop(0, C, add_body, 0)
        pltpu.sync_copy(bufv, out_hbm.at[pl.ds(off, C)])
        return carry

    lax.fori_loop(0, CHUNKS, chunk_body, 0)


def kernel(t, v, type_table, value_table):
    t_flat = jnp.reshape(t.astype(jnp.int32), (N,))
    v_flat = jnp.reshape(v.astype(jnp.int32), (N,))
    out = _sol_embedding(t_flat, v_flat, type_table, value_table)
    return jnp.reshape(out, (B, L, D))


# speedup vs baseline: 2.5340x; 1.2338x over previous
import functools

import jax
import jax.numpy as jnp
from jax import lax
from jax.experimental import pallas as pl
from jax.experimental.pallas import tpu as pltpu
from jax.experimental.pallas import tpu_sc as plsc

B, L, D = 4096, 200, 64
N = B * L
NC, NS = 2, 16
NW = NC * NS
PER_W = N // NW
C = 128
CHUNKS = PER_W // C
NBUF = 4
OUTER = CHUNKS // NBUF

_mesh = plsc.VectorSubcoreMesh(core_axis_name="c", subcore_axis_name="s")


@functools.partial(
    pl.kernel,
    mesh=_mesh,
    out_type=jax.ShapeDtypeStruct((N, D), jnp.float32),
    compiler_params=pltpu.CompilerParams(use_tc_tiling_on_sc=False),
    scratch_types=(
        [pltpu.VMEM((C,), jnp.int32) for _ in range(NBUF)]
        + [pltpu.VMEM((C,), jnp.int32) for _ in range(NBUF)]
        + [pltpu.VMEM((C, D), jnp.float32) for _ in range(NBUF)]
        + [pltpu.VMEM((C, D), jnp.float32) for _ in range(NBUF)]
        + [pltpu.SemaphoreType.DMA for _ in range(3 * NBUF)]
    ),
)
def _sol_embedding(t_hbm, v_hbm, tt_hbm, vt_hbm, out_hbm, *scr):
    ti = scr[0:NBUF]
    vi = scr[NBUF:2 * NBUF]
    bufv = scr[2 * NBUF:3 * NBUF]
    buft = scr[3 * NBUF:4 * NBUF]
    semi = scr[4 * NBUF:5 * NBUF]
    semg = scr[5 * NBUF:6 * NBUF]
    semo = scr[6 * NBUF:7 * NBUF]

    wid = lax.axis_index("s") * NC + lax.axis_index("c")
    base = wid * PER_W

    def issue_idx(g, s):
        off = base + g * C
        pltpu.async_copy(t_hbm.at[pl.ds(off, C)], ti[s], semi[s])
        pltpu.async_copy(v_hbm.at[pl.ds(off, C)], vi[s], semi[s])

    def wait_idx(s):
        pltpu.make_async_copy(t_hbm.at[pl.ds(0, C)], ti[s], semi[s]).wait()
        pltpu.make_async_copy(v_hbm.at[pl.ds(0, C)], vi[s], semi[s]).wait()

    def start_gathers(s):
        pltpu.async_copy(vt_hbm.at[vi[s]], bufv[s], semg[s])
        pltpu.async_copy(tt_hbm.at[ti[s]], buft[s], semg[s])

    def wait_gathers(s):
        pltpu.make_async_copy(vt_hbm.at[vi[s]], bufv[s], semg[s]).wait()
        pltpu.make_async_copy(tt_hbm.at[ti[s]], buft[s], semg[s]).wait()

    def start_write(g, s):
        off = base + g * C
        pltpu.async_copy(bufv[s], out_hbm.at[pl.ds(off, C)], semo[s])

    def wait_write(s):
        pltpu.make_async_copy(bufv[s], out_hbm.at[pl.ds(0, C)], semo[s]).wait()

    def add_rows(s):
        bv, bt = bufv[s], buft[s]

        def body(r, c):
            for rr in range(4):
                i = r * 4 + rr
                for j in range(D // 16):
                    x = bt[i, pl.ds(j * 16, 16)]
                    plsc.addupdate(bv.at[i, pl.ds(j * 16, 16)], x)
            return c

        lax.fori_loop(0, C // 4, body, 0)

    pltpu.sync_copy(t_hbm.at[pl.ds(base, C)], ti[0])
    pltpu.sync_copy(v_hbm.at[pl.ds(base, C)], vi[0])
    pltpu.sync_copy(t_hbm.at[pl.ds(base + C, C)], ti[1])
    pltpu.sync_copy(v_hbm.at[pl.ds(base + C, C)], vi[1])
    start_gathers(0)
    start_gathers(1)
    issue_idx(2, 2)
    issue_idx(3, 3)

    def outer(k, carry):
        for b in range(NBUF):
            g = k * NBUF + b
            s2 = (b + 2) % NBUF

            def stage_a(write_exists):
                wait_idx(s2)
                if write_exists:
                    wait_write(s2)
                else:
                    @pl.when(k >= 1)
                    def _():
                        wait_write(s2)
                start_gathers(s2)

            if b < 2:
                stage_a(write_exists=False)
            else:
                @pl.when(k < OUTER - 1)
                def _():
                    stage_a(write_exists=True)

            wait_gathers(b)

            @pl.when(k < OUTER - 1)
            def _():
                issue_idx(g + NBUF, b)

            add_rows(b)
            start_write(g, b)
        return carry

    lax.fori_loop(0, OUTER, outer, 0)

    for s in range(NBUF):
        wait_write(s)


def kernel(t, v, type_table, value_table):
    t_flat = jnp.reshape(t.astype(jnp.int32), (N,))
    v_flat = jnp.reshape(v.astype(jnp.int32), (N,))
    out = _sol_embedding(t_flat, v_flat, type_table, value_table)
    return jnp.reshape(out, (B, L, D))
